# baseline (speedup 1.0000x reference)
import functools

import jax
import jax.numpy as jnp
from jax.experimental import pallas as pl
from jax.experimental.pallas import tpu as pltpu

_CH = 256
_NBUF = 8
_NOB = 4


def _gate_kernel(chunks_per_batch, x_hbm, w_ref, b_ref, o_hbm,
                 xbuf, obuf, in_sem, out_sem):
    n_ch = x_hbm.shape[0] // _CH

    def in_copy(i, b):
        return pltpu.make_async_copy(
            x_hbm.at[pl.ds(i * _CH, _CH), :], xbuf.at[b], in_sem.at[b])

    def out_copy(i, b):
        return pltpu.make_async_copy(
            obuf.at[b], o_hbm.at[pl.ds(i * _CH, _CH), :], out_sem.at[b])

    for b in range(min(_NBUF, n_ch)):
        in_copy(b, b).start()

    for i in range(n_ch):
        b = i % _NBUF
        ob = i % _NOB
        in_copy(i, b).wait()
        s = jax.lax.dot_general(
            w_ref[...],
            xbuf[b],
            dimension_numbers=(((1,), (1,)), ((), ())),
            preferred_element_type=jnp.float32,
        )
        e = jnp.exp(s + b_ref[i // chunks_per_batch])
        o = e / jnp.sum(e, axis=0, keepdims=True)
        if i >= _NOB:
            out_copy(i - _NOB, ob).wait()
        obuf[ob] = o.T
        out_copy(i, ob).start()
        nxt = i + _NBUF
        if nxt < n_ch:
            in_copy(nxt, b).start()

    for i in range(max(0, n_ch - _NOB), n_ch):
        out_copy(i, i % _NOB).wait()


def kernel(x, W, train, active_experts):
    n_tok, d = x.shape
    n_exp = W.shape[0]
    batch = active_experts.shape[0]
    seq = n_tok // batch
    bias = jnp.where(active_experts > 0, 0.0, -jnp.inf).astype(jnp.float32)
    bias = bias.reshape(batch, n_exp, 1)

    out = pl.pallas_call(
        functools.partial(_gate_kernel, seq // _CH),
        in_specs=[
            pl.BlockSpec(memory_space=pltpu.MemorySpace.HBM),
            pl.BlockSpec(memory_space=pltpu.MemorySpace.VMEM),
            pl.BlockSpec(memory_space=pltpu.MemorySpace.VMEM),
        ],
        out_specs=pl.BlockSpec(memory_space=pltpu.MemorySpace.HBM),
        out_shape=jax.ShapeDtypeStruct((n_tok, n_exp), jnp.float32),
        scratch_shapes=[
            pltpu.VMEM((_NBUF, _CH, d), jnp.float32),
            pltpu.VMEM((_NOB, _CH, n_exp), jnp.float32),
            pltpu.SemaphoreType.DMA((_NBUF,)),
            pltpu.SemaphoreType.DMA((_NOB,)),
        ],
    )(x, W, bias)
    return out.reshape(batch, seq, n_exp)


# speedup vs baseline: 1.1699x; 1.1699x over previous
import jax
import jax.numpy as jnp
from jax.experimental import pallas as pl
from jax.experimental.pallas import tpu as pltpu

_H = 256
_SPLIT = 8
_BLK = _H * _SPLIT


def _gate_kernel(*refs):
    x_refs = refs[:_SPLIT]
    w_ref, b_ref, o_ref = refs[_SPLIT:]
    for k, x_ref in enumerate(x_refs):
        s = jax.lax.dot_general(
            w_ref[...],
            x_ref[...],
            dimension_numbers=(((1,), (1,)), ((), ())),
            preferred_element_type=jnp.float32,
        )
        s = s + b_ref[0]
        m = jnp.max(s, axis=0, keepdims=True)
        e = jnp.exp(s - m)
        o = e / jnp.sum(e, axis=0, keepdims=True)
        o_ref[k * _H:(k + 1) * _H, :] = o.T


def kernel(x, W, train, active_experts):
    n_tok, d = x.shape
    n_exp = W.shape[0]
    batch = active_experts.shape[0]
    seq = n_tok // batch
    blocks_per_batch = seq // _BLK
    bias = jnp.where(active_experts > 0, 0.0, -jnp.inf).astype(jnp.float32)
    bias = bias.reshape(batch, n_exp, 1)

    def x_spec(k):
        return pl.BlockSpec((_H, d), lambda i, k=k: (_SPLIT * i + k, 0))

    out = pl.pallas_call(
        _gate_kernel,
        grid=(n_tok // _BLK,),
        in_specs=[x_spec(k) for k in range(_SPLIT)] + [
            pl.BlockSpec((n_exp, d), lambda i: (0, 0)),
            pl.BlockSpec((1, n_exp, 1), lambda i: (i // blocks_per_batch, 0, 0)),
        ],
        out_specs=pl.BlockSpec((_BLK, n_exp), lambda i: (i, 0)),
        out_shape=jax.ShapeDtypeStruct((n_tok, n_exp), jnp.float32),
        compiler_params=pltpu.CompilerParams(dimension_semantics=("parallel",)),
    )(*([x] * _SPLIT), W, bias)
    return out.reshape(batch, seq, n_exp)
